# Initial kernel scaffold; baseline (speedup 1.0000x reference)
import functools

import jax
import jax.numpy as jnp
from jax import lax
from jax.experimental import pallas as pl
from jax.experimental.pallas import tpu as pltpu
import jax.experimental.pallas.tpu_sc as plsc

N = 10000
E = 320000
D = 128
W = 144
NC = 2
NS = 16
NW = NC * NS
EPW = E // NW
C = 80
NCHUNK = EPW // C
RPT = N // NS


def _pre_body(x_ref, beta_ref, xn_ref, nrm_ref, ws_ref):
    xb = x_ref[...]
    b = beta_ref[0, 0]
    n2 = jnp.sum(xb * xb, axis=1, keepdims=True)
    nrm = jnp.sqrt(n2)
    xn = xb / jnp.maximum(nrm, 1e-12)
    s2 = jnp.sum(xn * xn, axis=1, keepdims=True)
    ws = jnp.exp(b * s2 - jnp.abs(b))
    xn_ref[...] = xn
    nrm_ref[...] = nrm
    ws_ref[...] = ws


def _post_body(nf_ref, x_ref, ws_ref, out_ref):
    nf = nf_ref[...]
    msg = nf[0, :, 0:D] + nf[1, :, 0:D]
    den = nf[0, :, D:D + 1] + nf[1, :, D:D + 1]
    ws = ws_ref[...]
    xb = x_ref[...]
    out_ref[...] = jnp.tanh((msg + ws * xb) / (den + ws + 1e-16))


def _sc_body(src_hbm, dst_hbm, xn_hbm, nrm_hbm, prm_hbm, out_hbm,
             si_v, di_v, a_v, b_v, s_v, nrm_v, prm_v, tmp_v,
             num_sh, sem1, sem2):
    cid = lax.axis_index("c")
    sid = lax.axis_index("s")
    wid = cid * NS + sid

    zero16 = jnp.zeros((16,), jnp.float32)

    def zrow(r, carry):
        for k in range(W // 16):
            s_v[r, pl.ds(k * 16, 16)] = zero16
        return carry

    lax.fori_loop(0, C, zrow, 0)

    base_r = sid * RPT
    for j in range(RPT // C):
        pltpu.sync_copy(s_v, num_sh.at[pl.ds(base_r + j * C, C)])
    rem = RPT - (RPT // C) * C
    pltpu.sync_copy(s_v.at[pl.ds(0, rem)],
                    num_sh.at[pl.ds(base_r + (RPT // C) * C, rem)])

    pltpu.sync_copy(nrm_hbm, nrm_v)
    pltpu.sync_copy(prm_hbm, prm_v)
    plsc.subcore_barrier()

    ebase = wid * EPW

    def chunk_body(it, carry):
        base = ebase + it * C
        pltpu.sync_copy(src_hbm.at[pl.ds(base, C)], si_v)
        pltpu.sync_copy(dst_hbm.at[pl.ds(base, C)], di_v)
        pltpu.async_copy(xn_hbm.at[si_v], a_v, sem1).wait()
        pltpu.async_copy(xn_hbm.at[di_v], b_v, sem2).wait()
        beta_s = prm_v[0]
        c_s = prm_v[1]

        def edge_body(e, ecarry):
            al = [a_v[e, pl.ds(k * 16, 16)] for k in range(8)]
            bl = [b_v[e, pl.ds(k * 16, 16)] for k in range(8)]
            p0 = al[0] * bl[0] + al[1] * bl[1]
            p1 = al[2] * bl[2] + al[3] * bl[3]
            p2 = al[4] * bl[4] + al[5] * bl[5]
            p3 = al[6] * bl[6] + al[7] * bl[7]
            tot = (p0 + p1) + (p2 + p3)
            dot = jnp.sum(tot)
            dvec = jnp.broadcast_to(dot, (16,))
            wvec = jnp.exp(beta_s * dvec - c_s)
            nrm_s = nrm_v[si_v[e]]
            wp = wvec * nrm_s
            for k in range(8):
                s_v[e, pl.ds(k * 16, 16)] = al[k] * wp
            tmp_v[...] = wvec
            s_v[e, D] = tmp_v[0]
            return ecarry

        lax.fori_loop(0, C, edge_body, 0)
        pltpu.sync_copy(s_v, num_sh.at[di_v], add=True)
        return carry

    lax.fori_loop(0, NCHUNK, chunk_body, 0)
    plsc.subcore_barrier()

    out_base = cid * N + sid * RPT
    pltpu.sync_copy(num_sh.at[pl.ds(sid * RPT, RPT)],
                    out_hbm.at[pl.ds(out_base, RPT)])


_sc_edge = functools.partial(
    pl.kernel,
    out_type=jax.ShapeDtypeStruct((NC * N, W), jnp.float32),
    mesh=plsc.VectorSubcoreMesh(core_axis_name="c", subcore_axis_name="s"),
    scratch_types=[
        pltpu.VMEM((C,), jnp.int32),
        pltpu.VMEM((C,), jnp.int32),
        pltpu.VMEM((C, D), jnp.float32),
        pltpu.VMEM((C, D), jnp.float32),
        pltpu.VMEM((C, W), jnp.float32),
        pltpu.VMEM((N,), jnp.float32),
        pltpu.VMEM((16,), jnp.float32),
        pltpu.VMEM((16,), jnp.float32),
        pltpu.VMEM_SHARED((N, W), jnp.float32),
        pltpu.SemaphoreType.DMA,
        pltpu.SemaphoreType.DMA,
    ],
)(_sc_body)


@jax.jit
def kernel(x, edge_index, beta):
    src = edge_index[0]
    dst = edge_index[1]

    xn, nrm, wself = pl.pallas_call(
        _pre_body,
        grid=(N // 8,),
        in_specs=[
            pl.BlockSpec((8, D), lambda i: (i, 0)),
            pl.BlockSpec((1, 1), lambda i: (0, 0)),
        ],
        out_specs=[
            pl.BlockSpec((8, D), lambda i: (i, 0)),
            pl.BlockSpec((8, 1), lambda i: (i, 0)),
            pl.BlockSpec((8, 1), lambda i: (i, 0)),
        ],
        out_shape=[
            jax.ShapeDtypeStruct((N, D), jnp.float32),
            jax.ShapeDtypeStruct((N, 1), jnp.float32),
            jax.ShapeDtypeStruct((N, 1), jnp.float32),
        ],
    )(x, beta.reshape(1, 1))

    params = jnp.concatenate(
        [beta, jnp.abs(beta), jnp.zeros((14,), jnp.float32)])

    numfull = _sc_edge(src, dst, xn, nrm.reshape(-1), params)

    out = pl.pallas_call(
        _post_body,
        grid=(N // 8,),
        in_specs=[
            pl.BlockSpec((NC, 8, W), lambda i: (0, i, 0)),
            pl.BlockSpec((8, D), lambda i: (i, 0)),
            pl.BlockSpec((8, 1), lambda i: (i, 0)),
        ],
        out_specs=pl.BlockSpec((8, D), lambda i: (i, 0)),
        out_shape=jax.ShapeDtypeStruct((N, D), jnp.float32),
    )(numfull.reshape(NC, N, W), x, wself)
    return out


# speedup vs baseline: 3.5634x; 3.5634x over previous
import functools

import jax
import jax.numpy as jnp
from jax import lax
from jax.experimental import pallas as pl
from jax.experimental.pallas import tpu as pltpu
import jax.experimental.pallas.tpu_sc as plsc

N = 10000
E = 320000
D = 128
W = 144
NC = 2
NS = 16
NW = NC * NS
EPW = E // NW
C = 80
NCHUNK = EPW // C
RPT = 624


def _pre_body(x_ref, beta_ref, xn_ref, xnw_ref, ws_ref):
    xb = x_ref[...]
    b = beta_ref[0, 0]
    n2 = jnp.sum(xb * xb, axis=1, keepdims=True)
    nrm = jnp.sqrt(n2)
    xn = xb / jnp.maximum(nrm, 1e-12)
    s2 = jnp.sum(xn * xn, axis=1, keepdims=True)
    ws = jnp.exp(b * s2 - jnp.abs(b))
    xn_ref[...] = xn
    xnw_ref[...] = jnp.concatenate(
        [xn, nrm, jnp.zeros((xb.shape[0], W - D - 1), jnp.float32)], axis=1)
    ws_ref[...] = ws


def _post_body(nf_ref, x_ref, ws_ref, out_ref):
    nf = nf_ref[...]
    msg = nf[0, :, 0:D] + nf[1, :, 0:D]
    den = nf[0, :, D:D + 1] + nf[1, :, D:D + 1]
    ws = ws_ref[...]
    xb = x_ref[...]
    out_ref[...] = jnp.tanh((msg + ws * xb) / (den + ws + 1e-16))


def _sc_body(src_hbm, dst_hbm, xn_hbm, xnw_hbm, prm_hbm, out_hbm,
             si_v, di_v, a_v, b_v, s_v, prm_v,
             num_sh, sem1, sem2):
    cid = lax.axis_index("c")
    sid = lax.axis_index("s")
    wid = cid * NS + sid

    zero16 = jnp.zeros((16,), jnp.float32)

    def zrow(r, carry):
        for k in range(W // 16):
            s_v[r, pl.ds(k * 16, 16)] = zero16
        return carry

    lax.fori_loop(0, C, zrow, 0)

    base_r = sid * RPT
    for j in range(RPT // C):
        pltpu.sync_copy(s_v, num_sh.at[pl.ds(base_r + j * C, C)])
    rem = RPT - (RPT // C) * C
    pltpu.sync_copy(s_v.at[pl.ds(0, rem)],
                    num_sh.at[pl.ds(base_r + (RPT // C) * C, rem)])

    @pl.when(sid == NS - 1)
    def _zero_tail():
        pltpu.sync_copy(s_v.at[pl.ds(0, N - NS * RPT)],
                        num_sh.at[pl.ds(NS * RPT, N - NS * RPT)])

    pltpu.sync_copy(prm_hbm, prm_v)
    plsc.subcore_barrier()

    prm_vec = prm_v[...]
    beta_s = prm_vec[0]
    c_s = prm_vec[1]
    lanes = lax.iota(jnp.int32, 16)
    mask0f = jnp.where(lanes == 0, 1.0, 0.0).astype(jnp.float32)

    ebase = wid * EPW

    def chunk_body(it, carry):
        base = ebase + it * C
        pltpu.sync_copy(src_hbm.at[pl.ds(base, C)], si_v)
        pltpu.sync_copy(dst_hbm.at[pl.ds(base, C)], di_v)
        pltpu.async_copy(xnw_hbm.at[si_v], a_v, sem1).wait()
        pltpu.async_copy(xn_hbm.at[di_v], b_v, sem2).wait()

        def edge_body(e, ecarry):
            al = [a_v[e, pl.ds(k * 16, 16)] for k in range(8)]
            bl = [b_v[e, pl.ds(k * 16, 16)] for k in range(8)]
            tail = a_v[e, pl.ds(D, 16)]
            p0 = al[0] * bl[0] + al[1] * bl[1]
            p1 = al[2] * bl[2] + al[3] * bl[3]
            p2 = al[4] * bl[4] + al[5] * bl[5]
            p3 = al[6] * bl[6] + al[7] * bl[7]
            dot = jnp.sum((p0 + p1) + (p2 + p3))
            dvec = jnp.broadcast_to(dot, (16,))
            wvec = jnp.exp(beta_s * dvec - c_s)
            wp = wvec * tail[0]
            for k in range(8):
                s_v[e, pl.ds(k * 16, 16)] = al[k] * wp
            s_v[e, pl.ds(D, 16)] = wvec * mask0f
            return ecarry

        lax.fori_loop(0, C, edge_body, 0)
        pltpu.sync_copy(s_v, num_sh.at[di_v], add=True)
        return carry

    lax.fori_loop(0, NCHUNK, chunk_body, 0)
    plsc.subcore_barrier()

    out_base = cid * N + sid * RPT
    pltpu.sync_copy(num_sh.at[pl.ds(sid * RPT, RPT)],
                    out_hbm.at[pl.ds(out_base, RPT)])

    @pl.when(sid == NS - 1)
    def _drain_tail():
        pltpu.sync_copy(num_sh.at[pl.ds(NS * RPT, N - NS * RPT)],
                        out_hbm.at[pl.ds(cid * N + NS * RPT, N - NS * RPT)])


_sc_edge = functools.partial(
    pl.kernel,
    out_type=jax.ShapeDtypeStruct((NC * N, W), jnp.float32),
    mesh=plsc.VectorSubcoreMesh(core_axis_name="c", subcore_axis_name="s"),
    compiler_params=pltpu.CompilerParams(
        use_tc_tiling_on_sc=False, needs_layout_passes=False),
    scratch_types=[
        pltpu.VMEM((C,), jnp.int32),
        pltpu.VMEM((C,), jnp.int32),
        pltpu.VMEM((C, W), jnp.float32),
        pltpu.VMEM((C, D), jnp.float32),
        pltpu.VMEM((C, W), jnp.float32),
        pltpu.VMEM((16,), jnp.float32),
        pltpu.VMEM_SHARED((N, W), jnp.float32),
        pltpu.SemaphoreType.DMA,
        pltpu.SemaphoreType.DMA,
    ],
)(_sc_body)


@jax.jit
def kernel(x, edge_index, beta):
    src = edge_index[0]
    dst = edge_index[1]

    xn, xnw, wself = pl.pallas_call(
        _pre_body,
        grid=(N // 8,),
        in_specs=[
            pl.BlockSpec((8, D), lambda i: (i, 0)),
            pl.BlockSpec((1, 1), lambda i: (0, 0)),
        ],
        out_specs=[
            pl.BlockSpec((8, D), lambda i: (i, 0)),
            pl.BlockSpec((8, W), lambda i: (i, 0)),
            pl.BlockSpec((8, 1), lambda i: (i, 0)),
        ],
        out_shape=[
            jax.ShapeDtypeStruct((N, D), jnp.float32),
            jax.ShapeDtypeStruct((N, W), jnp.float32),
            jax.ShapeDtypeStruct((N, 1), jnp.float32),
        ],
    )(x, beta.reshape(1, 1))

    params = jnp.concatenate(
        [beta, jnp.abs(beta), jnp.zeros((14,), jnp.float32)])

    numfull = _sc_edge(src, dst, xn, xnw, params)

    out = pl.pallas_call(
        _post_body,
        grid=(N // 8,),
        in_specs=[
            pl.BlockSpec((NC, 8, W), lambda i: (0, i, 0)),
            pl.BlockSpec((8, D), lambda i: (i, 0)),
            pl.BlockSpec((8, 1), lambda i: (i, 0)),
        ],
        out_specs=pl.BlockSpec((8, D), lambda i: (i, 0)),
        out_shape=jax.ShapeDtypeStruct((N, D), jnp.float32),
    )(numfull.reshape(NC, N, W), x, wself)
    return out
